# baseline (speedup 1.0000x reference)
import functools
import math

import jax
import jax.numpy as jnp
import numpy as np
from jax.experimental import pallas as pl
from jax.experimental.pallas import tpu as pltpu

_SMOOTHING = 0.1
_BN = 32
_K = 4


def _loss_body(eps, coef_g, c_row, cw, v, tgt_sref, tgt_ref, *refs):
    xs = refs[:_K]
    out_ref = refs[_K]
    i = pl.program_id(0)
    bn = xs[0].shape[0]
    last_w = v - (_K - 1) * cw
    t = tgt_ref[...]
    valid = t != 0
    s = jnp.zeros((bn, 1), jnp.float32)
    for k, x_ref in enumerate(xs):
        x = x_ref[...]
        if k == _K - 1:
            col = jax.lax.broadcasted_iota(jnp.int32, x.shape, 1)
            x = jnp.where(col < last_w, x, 0.0)
        s = s + jnp.sum(x, axis=1, keepdims=True)
    s = s - xs[0][:, 0:1]
    part = jnp.sum(jnp.where(valid, s, 0.0))
    cnt = jnp.sum(jnp.where(valid, 1.0, 0.0))

    lane = jax.lax.broadcasted_iota(jnp.int32, (1, 128), 1)
    gpart = jnp.float32(0.0)
    for r in range(bn):
        tr = tgt_sref[i * bn + r]
        loc = tr % cw
        start = pl.multiple_of((loc // 128) * 128, 128)
        sel = jnp.float32(0.0)
        for k, x_ref in enumerate(xs):
            w = x_ref[pl.ds(r, 1), pl.ds(start, 128)]
            gk = jnp.sum(jnp.where(lane == loc % 128, w, 0.0))
            sel += jnp.where(tr // cw == k, gk, 0.0)
        gpart += jnp.where(tr != 0, sel, 0.0)

    @pl.when(i == 0)
    def _():
        out_ref[0, 0] = 0.0

    out_ref[0, 0] += c_row * cnt - eps * part - coef_g * gpart


def kernel(pred, target):
    n, v = pred.shape
    cw = -(-v // (_K * 128)) * 128
    eps = _SMOOTHING / (v - 2)
    eps32 = float(np.float32(eps))
    c_row = (v - 2) * (eps32 * math.log(eps32)) + 0.9 * math.log(0.9)
    coef_g = (1.0 - _SMOOTHING) - eps

    tgt2d = target.reshape(n, 1)

    def _chunk_spec(k):
        return pl.BlockSpec((_BN, cw), lambda i, *_, _k=k: (i, _k))

    grid_spec = pltpu.PrefetchScalarGridSpec(
        num_scalar_prefetch=1,
        grid=(n // _BN,),
        in_specs=[pl.BlockSpec((_BN, 1), lambda i, *_: (i, 0))]
        + [_chunk_spec(k) for k in range(_K)],
        out_specs=pl.BlockSpec(
            (1, 1), lambda i, *_: (0, 0), memory_space=pltpu.SMEM
        ),
    )
    out = pl.pallas_call(
        functools.partial(_loss_body, eps, coef_g, c_row, cw, v),
        grid_spec=grid_spec,
        out_shape=jax.ShapeDtypeStruct((1, 1), jnp.float32),
    )(target, tgt2d, *([pred] * _K))
    return out[0, 0]
